# baseline (speedup 1.0000x reference)
import functools

import jax
import jax.numpy as jnp
from jax import lax
from jax.experimental import pallas as pl
from jax.experimental.pallas import tpu as pltpu
from jax.experimental.pallas import tpu_sc as plsc

_NC = 2
_NS = 16
_NW = _NC * _NS

_SUB = 128
_GRP = 1024


def _gather_rows(total: int, D: int, weight, flat_idx):
    b_per_w = total // _NW
    groups = b_per_w // _GRP
    mesh = plsc.VectorSubcoreMesh(core_axis_name="c", subcore_axis_name="s")

    @functools.partial(
        pl.kernel,
        out_type=jax.ShapeDtypeStruct((total, D), jnp.float32),
        mesh=mesh,
        scratch_types=[
            pltpu.VMEM((b_per_w,), jnp.int32),
            pltpu.VMEM((_GRP, D), jnp.float32),
            pltpu.SemaphoreType.DMA,
        ],
        compiler_params=pltpu.CompilerParams(use_tc_tiling_on_sc=False),
    )
    def k(table_hbm, idx_hbm, out_hbm, idx_v, rows_v, sem):
        wid = lax.axis_index("s") * _NC + lax.axis_index("c")
        base = wid * b_per_w
        pltpu.sync_copy(idx_hbm.at[pl.ds(base, b_per_w)], idx_v)

        def body(g, carry):
            off = g * _GRP
            copies = []
            for j in range(_GRP // _SUB):
                copies.append(pltpu.async_copy(
                    table_hbm.at[idx_v.at[pl.ds(off + j * _SUB, _SUB)]],
                    rows_v.at[pl.ds(j * _SUB, _SUB)],
                    sem,
                ))
            for c in copies:
                c.wait()
            pltpu.sync_copy(rows_v, out_hbm.at[pl.ds(base + off, _GRP)])
            return carry

        lax.fori_loop(0, groups, body, 0)

    return k(weight, flat_idx)


def kernel(x, weight):
    B, S = x.shape
    V, D = weight.shape
    total = B * S
    flat_idx = x.reshape(total).astype(jnp.int32)
    out = _gather_rows(total, D, weight, flat_idx)
    return out.reshape(B, S, D)


# speedup vs baseline: 1.0010x; 1.0010x over previous
import functools

import jax
import jax.numpy as jnp
from jax import lax
from jax.experimental import pallas as pl
from jax.experimental.pallas import tpu as pltpu
from jax.experimental.pallas import tpu_sc as plsc

_NC = 2
_NS = 16
_NW = _NC * _NS

_SUB = 128
_GRP = 256
_NBUF = 4


def _gather_rows(total: int, D: int, weight, flat_idx):
    b_per_w = total // _NW
    groups = b_per_w // _GRP
    T = groups // _NBUF
    mesh = plsc.VectorSubcoreMesh(core_axis_name="c", subcore_axis_name="s")

    @functools.partial(
        pl.kernel,
        out_type=jax.ShapeDtypeStruct((total, D), jnp.float32),
        mesh=mesh,
        scratch_types=[
            pltpu.VMEM((b_per_w,), jnp.int32),
            [pltpu.VMEM((_GRP, D), jnp.float32) for _ in range(_NBUF)],
            [pltpu.SemaphoreType.DMA for _ in range(_NBUF)],
            [pltpu.SemaphoreType.DMA for _ in range(_NBUF)],
        ],
        compiler_params=pltpu.CompilerParams(use_tc_tiling_on_sc=False),
    )
    def k(table_hbm, idx_hbm, out_hbm, idx_v, rows, sg, sw):
        wid = lax.axis_index("s") * _NC + lax.axis_index("c")
        base = wid * b_per_w
        pltpu.sync_copy(idx_hbm.at[pl.ds(base, b_per_w)], idx_v)

        def fire_gathers(g, b):
            off = g * _GRP
            return [
                pltpu.async_copy(
                    table_hbm.at[idx_v.at[pl.ds(off + j * _SUB, _SUB)]],
                    rows[b].at[pl.ds(j * _SUB, _SUB)],
                    sg[b],
                )
                for j in range(_GRP // _SUB)
            ]

        def wait_write(b):
            pltpu.make_async_copy(
                rows[b], out_hbm.at[pl.ds(base, _GRP)], sw[b]
            ).wait()

        def body(t, carry):
            @pl.when(t > 0)
            def _():
                for b in range(_NBUF):
                    wait_write(b)

            gathers = []
            for b in range(_NBUF):
                g = t * _NBUF + b
                gathers.append(fire_gathers(g, b))
            for b in range(_NBUF):
                g = t * _NBUF + b
                for c in gathers[b]:
                    c.wait()
                pltpu.async_copy(
                    rows[b], out_hbm.at[pl.ds(base + g * _GRP, _GRP)], sw[b]
                )
            return carry

        lax.fori_loop(0, T, body, 0)
        for b in range(_NBUF):
            wait_write(b)

    return k(weight, flat_idx)


def kernel(x, weight):
    B, S = x.shape
    V, D = weight.shape
    total = B * S
    flat_idx = x.reshape(total).astype(jnp.int32)
    out = _gather_rows(total, D, weight, flat_idx)
    return out.reshape(B, S, D)


# speedup vs baseline: 1.0094x; 1.0084x over previous
import functools

import jax
import jax.numpy as jnp
from jax import lax
from jax.experimental import pallas as pl
from jax.experimental.pallas import tpu as pltpu
from jax.experimental.pallas import tpu_sc as plsc

_NC = 2
_NS = 16
_NW = _NC * _NS

_RWS = 160
_GRP = 64
_NBUF = 4

_MESH = plsc.VectorSubcoreMesh(core_axis_name="c", subcore_axis_name="s")


def _widen(weight):
    V, D = weight.shape
    chunks = V // _RWS
    T = (chunks + _NW - 1) // _NW

    @functools.partial(
        pl.kernel,
        out_type=jax.ShapeDtypeStruct((V, 128), jnp.float32),
        mesh=_MESH,
        scratch_types=[
            [pltpu.VMEM((_RWS, D), jnp.float32) for _ in range(2)],
            [pltpu.VMEM((_RWS, 128), jnp.float32) for _ in range(2)],
            [pltpu.SemaphoreType.DMA for _ in range(2)],
            [pltpu.SemaphoreType.DMA for _ in range(2)],
        ],
    )
    def k1(w_hbm, wp_hbm, abuf, bbuf, sa, sb):
        wid = lax.axis_index("s") * _NC + lax.axis_index("c")

        def chunk_of(t):
            return wid + t * _NW

        def repack(b):
            def row(r, carry):
                for kk in range(D // 16):
                    sl = pl.ds(kk * 16, 16)
                    bbuf[b][r, sl] = abuf[b][r, sl]
                return carry
            lax.fori_loop(0, _RWS, row, 0)

        def body(u, carry):
            for b in range(2):
                t = 2 * u + b
                c = chunk_of(t)

                @pl.when((t >= 2) & (chunk_of(t - 2) < chunks))
                def _():
                    pltpu.make_async_copy(
                        bbuf[b], wp_hbm.at[pl.ds(0, _RWS)], sb[b]
                    ).wait()

                @pl.when(c < chunks)
                def _():
                    pltpu.async_copy(
                        w_hbm.at[pl.ds(c * _RWS, _RWS)], abuf[b], sa[b]
                    )
            for b in range(2):
                t = 2 * u + b
                c = chunk_of(t)

                @pl.when(c < chunks)
                def _():
                    pltpu.make_async_copy(
                        w_hbm.at[pl.ds(0, _RWS)], abuf[b], sa[b]
                    ).wait()
                    repack(b)
                    pltpu.async_copy(
                        bbuf[b], wp_hbm.at[pl.ds(c * _RWS, _RWS)], sb[b]
                    )
            return carry

        lax.fori_loop(0, (T + 1) // 2, body, 0)
        for b in range(2):
            t = 2 * ((T + 1) // 2) - 2 + b

            @pl.when(chunk_of(t) < chunks)
            def _():
                pltpu.make_async_copy(
                    bbuf[b], wp_hbm.at[pl.ds(0, _RWS)], sb[b]
                ).wait()

    return k1(weight)


def _gather_rows(total: int, D: int, wlin, flat_idx):
    b_per_w = total // _NW
    groups = b_per_w // _GRP
    T = groups // _NBUF

    @functools.partial(
        pl.kernel,
        out_type=jax.ShapeDtypeStruct((total, D), jnp.float32),
        mesh=_MESH,
        scratch_types=[
            pltpu.VMEM((b_per_w,), jnp.int32),
            [pltpu.VMEM((_GRP, 128), jnp.float32) for _ in range(_NBUF)],
            [pltpu.VMEM((_GRP, D), jnp.float32) for _ in range(_NBUF)],
            [pltpu.SemaphoreType.DMA for _ in range(_NBUF)],
            [pltpu.SemaphoreType.DMA for _ in range(_NBUF)],
        ],
    )
    def k2(wlin_hbm, idx_hbm, out_hbm, idx_v, rows, cbuf, sg, sw):
        wid = lax.axis_index("s") * _NC + lax.axis_index("c")
        base = wid * b_per_w
        pltpu.sync_copy(idx_hbm.at[pl.ds(base, b_per_w)], idx_v)

        def repack(b):
            def row(r, carry):
                for kk in range(D // 16):
                    sl = pl.ds(kk * 16, 16)
                    cbuf[b][r, sl] = rows[b][r, sl]
                return carry
            lax.fori_loop(0, _GRP, row, 0)

        def wait_write(b):
            pltpu.make_async_copy(
                cbuf[b], out_hbm.at[pl.ds(base, _GRP)], sw[b]
            ).wait()

        def body(t, carry):
            @pl.when(t > 0)
            def _():
                for b in range(_NBUF):
                    wait_write(b)

            copies = []
            for b in range(_NBUF):
                g = t * _NBUF + b
                copies.append(pltpu.async_copy(
                    wlin_hbm.at[idx_v.at[pl.ds(g * _GRP, _GRP)]],
                    rows[b],
                    sg[b],
                ))
            for b in range(_NBUF):
                g = t * _NBUF + b
                copies[b].wait()
                repack(b)
                pltpu.async_copy(
                    cbuf[b], out_hbm.at[pl.ds(base + g * _GRP, _GRP)], sw[b]
                )
            return carry

        lax.fori_loop(0, T, body, 0)
        for b in range(_NBUF):
            wait_write(b)

    return k2(wlin, flat_idx)


def kernel(x, weight):
    B, S = x.shape
    V, D = weight.shape
    total = B * S
    wlin = _widen(weight)
    flat_idx = x.reshape(total).astype(jnp.int32)
    out = _gather_rows(total, D, wlin, flat_idx)
    return out.reshape(B, S, D)


# speedup vs baseline: 1.0603x; 1.0504x over previous
import functools

import jax
import jax.numpy as jnp
from jax import lax
from jax.experimental import pallas as pl
from jax.experimental.pallas import tpu as pltpu
from jax.experimental.pallas import tpu_sc as plsc

_NC = 2
_NS = 16
_NW = _NC * _NS

_NBUF = 2

_MESH = plsc.VectorSubcoreMesh(core_axis_name="c", subcore_axis_name="s")


def _gather_rows(B: int, S: int, D: int, wlin, flat_idx):
    total = B * S
    b_per_w = total // _NW
    nb_w = B // _NW
    mesh = _MESH

    @functools.partial(
        pl.kernel,
        out_type=jax.ShapeDtypeStruct((B, S, D), jnp.float32),
        mesh=mesh,
        scratch_types=[
            pltpu.VMEM((b_per_w,), jnp.int32),
            [pltpu.VMEM((S, 128), jnp.float32) for _ in range(_NBUF)],
            [pltpu.VMEM((1, S, D), jnp.float32) for _ in range(_NBUF)],
            [pltpu.SemaphoreType.DMA for _ in range(_NBUF)],
            [pltpu.SemaphoreType.DMA for _ in range(_NBUF)],
        ],
    )
    def k2(wlin_hbm, idx_hbm, out_hbm, idx_v, rows, cbuf, sg, sw):
        wid = lax.axis_index("s") * _NC + lax.axis_index("c")
        base = wid * b_per_w
        bbase = wid * nb_w
        pltpu.sync_copy(idx_hbm.at[pl.ds(base, b_per_w)], idx_v)

        def fire_gathers(g, b):
            off = g * S
            cps = []
            done = 0
            while done < S:
                n = min(128, S - done)
                cps.append(pltpu.async_copy(
                    wlin_hbm.at[idx_v.at[pl.ds(off + done, n)]],
                    rows[b].at[pl.ds(done, n)],
                    sg[b],
                ))
                done += n
            return cps

        def repack(b):
            def row(r, carry):
                for kk in range(D // 16):
                    sl = pl.ds(kk * 16, 16)
                    cbuf[b][0, r, sl] = rows[b][r, sl]
                return carry
            lax.fori_loop(0, S, row, 0)

        def wait_write(b):
            pltpu.make_async_copy(
                cbuf[b], out_hbm.at[pl.ds(bbase, 1)], sw[b]
            ).wait()

        def body(t, carry):
            @pl.when(t > 0)
            def _():
                for b in range(_NBUF):
                    wait_write(b)

            copies = []
            for b in range(_NBUF):
                copies.append(fire_gathers(t * _NBUF + b, b))
            for b in range(_NBUF):
                g = t * _NBUF + b
                for c in copies[b]:
                    c.wait()
                repack(b)
                pltpu.async_copy(
                    cbuf[b], out_hbm.at[pl.ds(bbase + g, 1)], sw[b]
                )
            return carry

        lax.fori_loop(0, nb_w // _NBUF, body, 0)
        for b in range(_NBUF):
            wait_write(b)

    return k2(wlin, flat_idx)


def kernel(x, weight):
    B, S = x.shape
    V, D = weight.shape
    wlin = jnp.pad(weight, ((0, 0), (0, 128 - D)))
    flat_idx = x.reshape(B * S).astype(jnp.int32)
    return _gather_rows(B, S, D, wlin, flat_idx)
